# baseline (speedup 1.0000x reference)
import functools

import jax
import jax.numpy as jnp
from jax import lax
from jax.experimental import pallas as pl
from jax.experimental.pallas import tpu as pltpu
from jax.experimental.pallas import tpu_sc as plsc

N_FIELDS = 26
VOCAB = 100000
D = 32
B = 16384

NC = 2
NS = 16
NW = NC * NS

R = B * N_FIELDS
RPW = R // NW
C = 832
CPW = RPW // C

_mesh = plsc.VectorSubcoreMesh(core_axis_name="c", subcore_axis_name="s")


@functools.partial(
    pl.kernel,
    mesh=_mesh,
    compiler_params=pltpu.CompilerParams(use_tc_tiling_on_sc=False),
    out_type=jax.ShapeDtypeStruct((R, D), jnp.float32),
    scratch_types=[
        pltpu.VMEM((C,), jnp.int32),
        pltpu.VMEM((C,), jnp.int32),
        pltpu.VMEM((C, D), jnp.float32),
        pltpu.VMEM((N_FIELDS * D,), jnp.float32),
        pltpu.SemaphoreType.DMA,
    ],
)
def _emb_lookup(xf_hbm, tab_hbm, pat_hbm, bias_hbm, out_hbm,
                idx_v, pat_v, rows_v, bias_v, sem):
    wid = lax.axis_index("s") * NC + lax.axis_index("c")
    pltpu.sync_copy(bias_hbm, bias_v)
    pltpu.sync_copy(pat_hbm, pat_v)
    w0 = wid * RPW

    def chunk_body(ch, carry):
        r0 = w0 + ch * C

        pltpu.sync_copy(xf_hbm.at[pl.ds(r0, C)], idx_v)

        def add_off(i, carry):
            sl = pl.ds(i * 16, 16)
            idx_v[sl] = idx_v[sl] + pat_v[sl]
            return carry

        lax.fori_loop(0, C // 16, add_off, 0)

        pltpu.async_copy(tab_hbm.at[idx_v], rows_v, sem).wait()

        def add_bias_period(q, carry):
            def add_bias_row(ff, carry):
                r = q * N_FIELDS + ff
                b0 = ff * D
                rows_v[r, pl.ds(0, 16)] = (
                    rows_v[r, pl.ds(0, 16)] + bias_v[pl.ds(b0, 16)])
                rows_v[r, pl.ds(16, 16)] = (
                    rows_v[r, pl.ds(16, 16)] + bias_v[pl.ds(b0 + 16, 16)])
                return carry

            return lax.fori_loop(0, N_FIELDS, add_bias_row, carry)

        lax.fori_loop(0, C // N_FIELDS, add_bias_period, 0)

        pltpu.sync_copy(rows_v, out_hbm.at[pl.ds(r0, C), :])
        return carry

    lax.fori_loop(0, CPW, chunk_body, 0)


def kernel(x, tables, bias):
    xf = x.reshape(R)
    tab2 = tables.reshape(N_FIELDS * VOCAB, D)
    pat = (jnp.arange(C, dtype=jnp.int32) % N_FIELDS) * VOCAB
    out2 = _emb_lookup(xf, tab2, pat, bias.reshape(N_FIELDS * D))
    return out2.reshape(B, N_FIELDS, D)


# speedup vs baseline: 1.1166x; 1.1166x over previous
import functools

import jax
import jax.numpy as jnp
from jax import lax
from jax.experimental import pallas as pl
from jax.experimental.pallas import tpu as pltpu
from jax.experimental.pallas import tpu_sc as plsc

N_FIELDS = 26
VOCAB = 100000
D = 32
B = 16384

NC = 2
NS = 16
NW = NC * NS

R = B * N_FIELDS
RPW = R // NW
C = 1664
CPW = RPW // C
QP = C // N_FIELDS

_mesh = plsc.VectorSubcoreMesh(core_axis_name="c", subcore_axis_name="s")


@functools.partial(
    pl.kernel,
    mesh=_mesh,
    compiler_params=pltpu.CompilerParams(use_tc_tiling_on_sc=False),
    out_type=jax.ShapeDtypeStruct((R, D), jnp.float32),
    scratch_types=[
        pltpu.VMEM((C,), jnp.int32),
        pltpu.VMEM((C,), jnp.int32),
        pltpu.VMEM((C,), jnp.int32),
        pltpu.VMEM((C, D), jnp.float32),
        pltpu.VMEM((C, D), jnp.float32),
        pltpu.VMEM((C,), jnp.int32),
        pltpu.VMEM((N_FIELDS * D,), jnp.float32),
        pltpu.SemaphoreType.DMA((8,)),
    ],
)
def _emb_lookup(xf_hbm, tab_hbm, pat_hbm, bias_hbm, out_hbm,
                idx0, idx1, idx2, rows0, rows1, pat_v, bias_v, sems):
    idx = [idx0, idx1, idx2]
    rows = [rows0, rows1]
    wid = lax.axis_index("s") * NC + lax.axis_index("c")
    w0 = wid * RPW
    pltpu.sync_copy(bias_hbm, bias_v)
    pltpu.sync_copy(pat_hbm, pat_v)

    b_lo = [bias_v[pl.ds(ff * D, 16)] for ff in range(N_FIELDS)]
    b_hi = [bias_v[pl.ds(ff * D + 16, 16)] for ff in range(N_FIELDS)]

    cp_idx = {}
    cp_g = {}
    cp_out = {}

    def start_idx(ch):
        s = ch % 3
        cp_idx[ch] = pltpu.async_copy(
            xf_hbm.at[pl.ds(w0 + ch * C, C)], idx[s], sems.at[s])

    def start_gather(ch):
        s = ch % 3
        cp_idx[ch].wait()
        iv = idx[s]

        def add_off(i, carry):
            sl = pl.ds(i * 16, 16)
            iv[sl] = iv[sl] + pat_v[sl]
            return carry

        lax.fori_loop(0, C // 16, add_off, 0)
        rs = ch % 2
        cp_g[ch] = pltpu.async_copy(tab_hbm.at[iv], rows[rs], sems.at[3 + rs])

    def finish(ch):
        rs = ch % 2
        cp_g[ch].wait()
        rv = rows[rs]

        def add_bias(q, carry):
            r0 = q * N_FIELDS
            for ff in range(N_FIELDS):
                rv[r0 + ff, pl.ds(0, 16)] = rv[r0 + ff, pl.ds(0, 16)] + b_lo[ff]
                rv[r0 + ff, pl.ds(16, 16)] = rv[r0 + ff, pl.ds(16, 16)] + b_hi[ff]
            return carry

        lax.fori_loop(0, QP, add_bias, 0)
        cp_out[ch] = pltpu.async_copy(
            rv, out_hbm.at[pl.ds(w0 + ch * C, C), :], sems.at[5 + rs])

    start_idx(0)
    start_gather(0)
    start_idx(1)
    for ch in range(CPW):
        if ch >= 1:
            cp_out[ch - 1].wait()
        if ch + 1 < CPW:
            start_gather(ch + 1)
        if ch + 2 < CPW:
            start_idx(ch + 2)
        finish(ch)
    cp_out[CPW - 1].wait()


def kernel(x, tables, bias):
    xf = x.reshape(R)
    tab2 = tables.reshape(N_FIELDS * VOCAB, D)
    pat = (jnp.arange(C, dtype=jnp.int32) % N_FIELDS) * VOCAB
    out2 = _emb_lookup(xf, tab2, pat, bias.reshape(N_FIELDS * D))
    return out2.reshape(B, N_FIELDS, D)


# speedup vs baseline: 1.7634x; 1.5792x over previous
import functools

import jax
import jax.numpy as jnp
from jax import lax
from jax.experimental import pallas as pl
from jax.experimental.pallas import tpu as pltpu
from jax.experimental.pallas import tpu_sc as plsc

N_FIELDS = 26
VOCAB = 100000
D = 32
B = 16384

HB = B // 2
UNR = 8
VC0 = 50048
VC1 = 49920
TB = VC0 + VC1
TL = VOCAB - TB

_mesh = plsc.VectorSubcoreMesh(core_axis_name="c", subcore_axis_name="s")


@functools.partial(
    pl.kernel,
    mesh=_mesh,
    compiler_params=pltpu.CompilerParams(use_tc_tiling_on_sc=True,
                                         needs_layout_passes=False),
    out_type=jax.ShapeDtypeStruct((N_FIELDS, D, B), jnp.float32),
    scratch_types=[
        pltpu.VMEM_SHARED((8, VOCAB), jnp.float32),
        pltpu.VMEM_SHARED((8, B), jnp.float32),
        pltpu.VMEM((VC0,), jnp.float32),
        pltpu.VMEM((8, TL), jnp.float32),
        pltpu.VMEM((HB,), jnp.int32),
        pltpu.VMEM((HB,), jnp.float32),
        pltpu.VMEM((D,), jnp.float32),
    ],
)
def _emb_lookup(xq_hbm, tt_hbm, tailq_hbm, biasq_hbm, out_hbm,
                stage_s, out_s, plane_v, tail_v, idx_v, res_v,
                bias_v):
    c = lax.axis_index("c")
    s = lax.axis_index("s")
    p = s % 8
    h = s // 8
    d_lo = pl.multiple_of(c * 16, 16)
    lane = lax.broadcasted_iota(jnp.int32, (16,), 0)

    def field_body(f, carry):
        pltpu.sync_copy(xq_hbm.at[f, h, :], idx_v)
        pltpu.sync_copy(biasq_hbm.at[f, 0, :], bias_v)
        bv16 = bias_v[pl.ds(d_lo % D, 16)]

        def group_body(j, carryj):
            d8 = pl.multiple_of(c * 16 + j * 8, 8)

            @pl.when(s == 0)
            def _():
                pltpu.sync_copy(tt_hbm.at[f, pl.ds(d8, 8), :], stage_s)

            plsc.subcore_barrier()

            pltpu.sync_copy(tailq_hbm.at[f, d8 + p, :, :], tail_v)

            bscalar = jnp.sum(jnp.where(lane == j * 8 + p, bv16, 0.0))
            bvec = jnp.full((16,), bscalar, dtype=jnp.float32)

            for k in range(2):
                vb = 0 if k == 0 else VC0
                vl = VC0 if k == 0 else VC1
                pltpu.sync_copy(stage_s.at[p, pl.ds(vb, vl)],
                                plane_v.at[pl.ds(0, vl)])

                def gather_blk(i, carry2):
                    for u in range(UNR):
                        o = (i * UNR + u) * 16
                        iv = idx_v[pl.ds(o, 16)]
                        if k == 0:
                            mask = iv < VC0
                            loc = jnp.where(mask, iv, 0)
                            g = plsc.load_gather(plane_v, [loc], mask=mask)
                            res_v[pl.ds(o, 16)] = g + bvec
                        else:
                            mask = (iv >= VC0) & (iv < TB)
                            loc = jnp.where(mask, iv - VC0, 0)
                            g = plsc.load_gather(plane_v, [loc], mask=mask)
                            mt = iv >= TB
                            lt = jnp.where(mt, iv - TB, 0)
                            z16 = jnp.zeros((16,), jnp.int32)
                            gt = plsc.load_gather(tail_v, [z16, lt], mask=mt)
                            r0 = res_v[pl.ds(o, 16)]
                            r1 = jnp.where(mask, g + bvec, r0)
                            res_v[pl.ds(o, 16)] = jnp.where(mt, gt + bvec, r1)
                    return carry2

                lax.fori_loop(0, HB // (16 * UNR), gather_blk, 0)

            pltpu.sync_copy(res_v, out_s.at[p, pl.ds(h * HB, HB)])
            plsc.subcore_barrier()

            @pl.when(s == 0)
            def _():
                pltpu.sync_copy(out_s, out_hbm.at[f, pl.ds(d8, 8), :])
            return carryj

        lax.fori_loop(0, 2, group_body, 0)
        return carry

    lax.fori_loop(0, N_FIELDS, field_body, 0)


def kernel(x, tables, bias):
    xq = jnp.zeros((N_FIELDS, 8, HB), jnp.int32).at[:, :2, :].set(
        x.T.reshape(N_FIELDS, 2, HB))
    biasq = jnp.zeros((N_FIELDS, 8, D), jnp.float32).at[:, 0, :].set(bias)
    tt = tables.transpose(0, 2, 1)
    tail = lax.slice(tt, (0, 0, TB), (N_FIELDS, D, VOCAB))
    tailq = jnp.zeros((N_FIELDS, D, 8, TL), jnp.float32).at[:, :, 0, :].set(tail)
    out_t = _emb_lookup(xq, tt, tailq, biasq)
    return out_t.transpose(2, 0, 1)


# speedup vs baseline: 1.7903x; 1.0153x over previous
import functools

import jax
import jax.numpy as jnp
from jax import lax
from jax.experimental import pallas as pl
from jax.experimental.pallas import tpu as pltpu
from jax.experimental.pallas import tpu_sc as plsc

N_FIELDS = 26
VOCAB = 100000
D = 32
B = 16384

HB = B // 2
UNR = 8
VC0 = 50048
VC1 = 49920
TB = VC0 + VC1
TL = VOCAB - TB

_mesh = plsc.VectorSubcoreMesh(core_axis_name="c", subcore_axis_name="s")


@functools.partial(
    pl.kernel,
    mesh=_mesh,
    compiler_params=pltpu.CompilerParams(use_tc_tiling_on_sc=True,
                                         needs_layout_passes=False),
    out_type=jax.ShapeDtypeStruct((N_FIELDS, D, B), jnp.float32),
    scratch_types=[
        pltpu.VMEM_SHARED((8, VOCAB), jnp.float32),
        pltpu.VMEM_SHARED((8, B), jnp.float32),
        pltpu.VMEM((VC0,), jnp.float32),
        pltpu.VMEM((8, TL), jnp.float32),
        pltpu.VMEM((HB,), jnp.int32),
        pltpu.VMEM((HB,), jnp.float32),
        pltpu.VMEM((D,), jnp.float32),
    ],
)
def _emb_lookup(xq_hbm, tt_hbm, tailq_hbm, biasq_hbm, out_hbm,
                stage_s, out_s, plane_v, tail_v, idx_v, res_v,
                bias_v):
    c = lax.axis_index("c")
    s = lax.axis_index("s")
    p = s % 8
    h = s // 8
    d_lo = pl.multiple_of(c * 16, 16)
    lane = lax.broadcasted_iota(jnp.int32, (16,), 0)

    def field_body(f, carry):
        pltpu.sync_copy(xq_hbm.at[f, h, :], idx_v)
        pltpu.sync_copy(biasq_hbm.at[f, 0, :], bias_v)
        bv16 = bias_v[pl.ds(d_lo % D, 16)]

        def group_body(j, carryj):
            d8 = pl.multiple_of(c * 16 + j * 8, 8)

            @pl.when(s == 0)
            def _():
                pltpu.sync_copy(tt_hbm.at[f, pl.ds(d8, 8), :], stage_s)

            plsc.subcore_barrier()

            pltpu.sync_copy(tailq_hbm.at[f, d8 + p, :, :], tail_v)

            bscalar = jnp.sum(jnp.where(lane == j * 8 + p, bv16, 0.0))
            bvec = jnp.full((16,), bscalar, dtype=jnp.float32)

            for k in range(2):
                vb = 0 if k == 0 else VC0
                vl = VC0 if k == 0 else VC1
                pltpu.sync_copy(stage_s.at[p, pl.ds(vb, vl)],
                                plane_v.at[pl.ds(0, vl)])
                if k == 1:
                    plane_v[pl.ds(VC1, 16)] = tail_v[0, pl.ds(0, 16)]
                    plane_v[pl.ds(VC1 + 16, 16)] = tail_v[0, pl.ds(16, 16)]

                def gather_blk(i, carry2):
                    for u in range(UNR):
                        o = (i * UNR + u) * 16
                        iv = idx_v[pl.ds(o, 16)]
                        if k == 0:
                            mask = iv < VC0
                            loc = jnp.where(mask, iv, 0)
                            g = plsc.load_gather(plane_v, [loc], mask=mask)
                            res_v[pl.ds(o, 16)] = g + bvec
                        else:
                            mask = iv >= VC0
                            loc = jnp.where(mask, iv - VC0, 0)
                            g = plsc.load_gather(plane_v, [loc], mask=mask)
                            r0 = res_v[pl.ds(o, 16)]
                            res_v[pl.ds(o, 16)] = jnp.where(mask, g + bvec, r0)
                    return carry2

                lax.fori_loop(0, HB // (16 * UNR), gather_blk, 0)

            pltpu.sync_copy(res_v, out_s.at[p, pl.ds(h * HB, HB)])
            plsc.subcore_barrier()

            @pl.when(s == 0)
            def _():
                pltpu.sync_copy(out_s, out_hbm.at[f, pl.ds(d8, 8), :])
            return carryj

        lax.fori_loop(0, 2, group_body, 0)
        return carry

    lax.fori_loop(0, N_FIELDS, field_body, 0)


def kernel(x, tables, bias):
    xq = jnp.zeros((N_FIELDS, 8, HB), jnp.int32).at[:, :2, :].set(
        x.T.reshape(N_FIELDS, 2, HB))
    biasq = jnp.zeros((N_FIELDS, 8, D), jnp.float32).at[:, 0, :].set(bias)
    tt = tables.transpose(0, 2, 1)
    tail = lax.slice(tt, (0, 0, TB), (N_FIELDS, D, VOCAB))
    tailq = jnp.zeros((N_FIELDS, D, 8, TL), jnp.float32).at[:, :, 0, :].set(tail)
    out_t = _emb_lookup(xq, tt, tailq, biasq)
    return out_t.transpose(2, 0, 1)
